# baseline (speedup 1.0000x reference)
import functools

import jax
import jax.numpy as jnp
from jax import lax
from jax.experimental import pallas as pl
from jax.experimental.pallas import tpu as pltpu
from jax.experimental.pallas import tpu_sc as plsc

D_MODEL = 64
SCALE = 8.0
NUM_WORKERS = 32
SOCT = 8
BLANE = 128


def kernel(x, table):
    s0, s1 = x.shape
    bsz = s0 * s1
    vocab = table.shape[0]
    xT = x.T.astype(jnp.int32)
    table2 = table.reshape(vocab // 2, 2 * D_MODEL)
    n_oct = s1 // SOCT
    n_bt = s0 // BLANE
    units = n_oct * n_bt
    per_w = units // NUM_WORKERS
    assert per_w >= 4
    stag_len = SOCT * D_MODEL * BLANE

    mesh = plsc.VectorSubcoreMesh(core_axis_name="c", subcore_axis_name="s")

    @functools.partial(
        pl.kernel,
        mesh=mesh,
        out_type=jax.ShapeDtypeStruct((bsz * D_MODEL,), jnp.float32),
        scratch_types=[
            pltpu.VMEM((2, SOCT, BLANE), jnp.int32),
            [pltpu.VMEM((BLANE,), jnp.int32) for _ in range(2)],
            [pltpu.VMEM((BLANE, 2 * D_MODEL), jnp.float32) for _ in range(2)],
            pltpu.VMEM((stag_len,), jnp.float32),
            [pltpu.SemaphoreType.DMA for _ in range(2)],
            pltpu.SemaphoreType.DMA,
            pltpu.SemaphoreType.DMA,
        ],
        compiler_params=pltpu.CompilerParams(
            use_tc_tiling_on_sc=True, needs_layout_passes=False
        ),
    )
    def emb(x_hbm, table_hbm, out_hbm, idxb, rowid, gath, stag, gsem, ssem, isem):
        wid = lax.axis_index("s") * 2 + lax.axis_index("c")
        u0 = wid * per_w
        u_last = u0 + per_w - 1
        lane = lax.iota(jnp.int32, 16)

        def unit_oct_bt(u):
            o = u // n_bt
            return o, u - o * n_bt

        def prefetch_idx(u, ib):
            o, bt = unit_oct_bt(u)
            src = x_hbm.at[pl.ds(o * SOCT, SOCT), pl.ds(bt * BLANE, BLANE)]
            pltpu.async_copy(src, idxb.at[ib], isem)

        def wait_idx():
            pltpu.make_async_copy(
                x_hbm.at[pl.ds(0, SOCT), pl.ds(0, BLANE)],
                idxb.at[0],
                isem,
            ).wait()

        def issue_gather(ib, s8, b):
            for i in range(BLANE // 16):
                sl = pl.ds(i * 16, 16)
                rowid[b][sl] = jax.lax.shift_right_logical(idxb[ib, s8, sl], 1)
            pltpu.async_copy(table_hbm.at[rowid[b]], gath[b], gsem[b])

        def wait_gather(b):
            pltpu.make_async_copy(
                table_hbm.at[pl.ds(0, BLANE)], gath[b], gsem[b]
            ).wait()

        def issue_scatters(u):
            o, bt = unit_oct_bt(u)
            for s8 in range(SOCT):
                for dt in range(D_MODEL // 8):
                    src = stag.at[pl.ds((s8 * 8 + dt) * 8 * BLANE, 8 * BLANE)]
                    off = ((o * SOCT + s8) * 8 + dt) * (8 * BLANE * n_bt) + bt * (
                        8 * BLANE
                    )
                    pltpu.async_copy(src, out_hbm.at[pl.ds(off, 8 * BLANE)], ssem)

        def wait_scatters():
            pltpu.make_async_copy(
                stag, out_hbm.at[pl.ds(0, stag_len)], ssem
            ).wait()

        def compact(ib, s8, b):
            for jg in range(BLANE // 16):
                rowv = jg * 16 + lane
                parv = (idxb[ib, s8, pl.ds(jg * 16, 16)] & 1) * D_MODEL

                @plsc.parallel_loop(0, D_MODEL, step=1)
                def _tr(d):
                    vals = plsc.load_gather(gath[b], [rowv, parv + d])
                    dst = s8 * (8 * D_MODEL * 16) + d * BLANE + jg * 16
                    stag[pl.ds(dst, 16)] = vals * SCALE

        def unit_body(u, ib, first, last):
            for s8 in range(SOCT):
                b = s8 % 2
                wait_gather(b)
                if s8 == 0 and not first:
                    wait_scatters()
                compact(ib, s8, b)
                if s8 < SOCT - 2:
                    issue_gather(ib, s8 + 2, b)
                elif not last:
                    if s8 == SOCT - 2:
                        wait_idx()
                    issue_gather(1 - ib, s8 - (SOCT - 2), b)
            issue_scatters(u)
            if not last:
                prefetch_idx(lax.min(u + 2, u_last), ib)

        pltpu.sync_copy(
            x_hbm.at[
                pl.ds((u0 // n_bt) * SOCT, SOCT),
                pl.ds((u0 % n_bt) * BLANE, BLANE),
            ],
            idxb.at[0],
        )
        prefetch_idx(u0 + 1, 1)
        issue_gather(0, 0, 0)
        issue_gather(0, 1, 1)

        unit_body(u0, 0, True, False)

        def steady(k, carry):
            unit_body(u0 + k, k % 2, False, False)
            return carry

        lax.fori_loop(1, per_w - 1, steady, 0)

        unit_body(u_last, (per_w - 1) % 2, False, True)

        wait_idx()
        wait_scatters()

    out = emb(xT, table2)
    out5 = out.reshape(s1, D_MODEL // 8, n_bt, 8, BLANE)
    out3 = out5.transpose(2, 4, 0, 1, 3).reshape(s0, s1, D_MODEL)
    return out3


# speedup vs baseline: 1.5612x; 1.5612x over previous
import functools

import jax
import jax.numpy as jnp
from jax import lax
from jax.experimental import pallas as pl
from jax.experimental.pallas import tpu as pltpu
from jax.experimental.pallas import tpu_sc as plsc

D_MODEL = 64
SCALE = 8.0
NUM_WORKERS = 32
CHUNK = 128


def kernel(x, table):
    s0, s1 = x.shape
    bsz = s0 * s1
    vocab = table.shape[0]
    xf = x.reshape(bsz).astype(jnp.int32)
    table2 = table.reshape(vocab // 2, 2 * D_MODEL)
    b_per_w = bsz // NUM_WORKERS
    n_chunks = b_per_w // CHUNK
    assert n_chunks % 2 == 0 and n_chunks >= 4

    mesh = plsc.VectorSubcoreMesh(core_axis_name="c", subcore_axis_name="s")

    @functools.partial(
        pl.kernel,
        mesh=mesh,
        out_type=jax.ShapeDtypeStruct((bsz, D_MODEL), jnp.float32),
        scratch_types=[
            pltpu.VMEM((b_per_w,), jnp.int32),
            [pltpu.VMEM((CHUNK,), jnp.int32) for _ in range(2)],
            [pltpu.VMEM((CHUNK, 2 * D_MODEL), jnp.float32) for _ in range(2)],
            [pltpu.VMEM((CHUNK, D_MODEL), jnp.float32) for _ in range(2)],
            [pltpu.SemaphoreType.DMA for _ in range(2)],
            [pltpu.SemaphoreType.DMA for _ in range(2)],
        ],
        compiler_params=pltpu.CompilerParams(
            use_tc_tiling_on_sc=True, needs_layout_passes=False
        ),
    )
    def emb(x_hbm, table_hbm, out_hbm, idx_v, rowid, gath, outb, gsem, ssem):
        wid = lax.axis_index("s") * 2 + lax.axis_index("c")
        base = wid * b_per_w
        lane = lax.iota(jnp.int32, 16)

        def start_gather(g, b):
            for i in range(CHUNK // 16):
                sl = pl.ds(g * CHUNK + i * 16, 16)
                rowid[b][pl.ds(i * 16, 16)] = jax.lax.shift_right_logical(
                    idx_v[sl], 1
                )
            pltpu.async_copy(table_hbm.at[rowid[b]], gath[b], gsem[b])

        def wait_gather(b):
            pltpu.make_async_copy(
                table_hbm.at[pl.ds(0, CHUNK)], gath[b], gsem[b]
            ).wait()

        def issue_scatter(g, b):
            dst = out_hbm.at[pl.ds(base + g * CHUNK, CHUNK)]
            pltpu.async_copy(outb[b], dst, ssem[b])

        def wait_scatter(b):
            pltpu.make_async_copy(
                outb[b], out_hbm.at[pl.ds(0, CHUNK)], ssem[b]
            ).wait()

        def turn(g, b, first, last):
            if not first:
                wait_scatter(b)
            wait_gather(b)

            @plsc.parallel_loop(0, CHUNK, step=16)
            def _compact(r0):
                par = (idx_v[pl.ds(g * CHUNK + r0, 16)] & 1) * D_MODEL

                for j in range(16):
                    r = r0 + j
                    rowv = jax.lax.broadcast(r, (16,))
                    src_half = par[j]
                    for c in range(D_MODEL // 16):
                        colv = src_half + c * 16 + lane
                        vals = plsc.load_gather(gath[b], [rowv, colv])
                        outb[b][r, pl.ds(c * 16, 16)] = vals * SCALE

            issue_scatter(g, b)
            if not last:
                start_gather(g + 2, b)

        pltpu.sync_copy(x_hbm.at[pl.ds(base, b_per_w)], idx_v)
        start_gather(0, 0)
        start_gather(1, 1)

        turn(0, 0, True, False)
        turn(1, 1, True, False)

        def cycle(gg, carry):
            turn(2 * gg, 0, False, False)
            turn(2 * gg + 1, 1, False, False)
            return carry

        lax.fori_loop(1, n_chunks // 2 - 1, cycle, 0)

        turn(n_chunks - 2, 0, False, True)
        turn(n_chunks - 1, 1, False, True)

        wait_scatter(0)
        wait_scatter(1)

    out = emb(xf, table2)
    return out.reshape(s0, s1, D_MODEL)
